# baseline (speedup 1.0000x reference)
import functools

import jax
import jax.numpy as jnp
from jax import lax
from jax.experimental import pallas as pl
from jax.experimental.pallas import tpu as pltpu
from jax.experimental.pallas import tpu_sc as plsc

N_ = 10000
E_ = 320000
D_ = 128
NS_ = 0.5

NPAD = 10240
CHUNK = 80
GRP = 8
PERW = 10240
RBLK = 1000

_NC = 2
_NSS = 16


def _leaky(h):
    return jnp.where(h >= 0.0, h, NS_ * h)


def _mm_stage(x, Wl, Wr, b):
    def body(x_ref, wl_ref, wr_ref, b_ref, z_ref, r_ref):
        xv = x_ref[...]
        z_ref[...] = jnp.dot(xv, wl_ref[...], preferred_element_type=jnp.float32)
        r_ref[...] = (
            jnp.dot(xv, wr_ref[...], preferred_element_type=jnp.float32)
            + b_ref[...]
        )

    return pl.pallas_call(
        body,
        grid=(N_ // RBLK,),
        in_specs=[
            pl.BlockSpec((RBLK, D_), lambda n: (n, 0)),
            pl.BlockSpec((D_, D_), lambda n: (0, 0)),
            pl.BlockSpec((D_, D_), lambda n: (0, 0)),
            pl.BlockSpec((1, D_), lambda n: (0, 0)),
        ],
        out_specs=[
            pl.BlockSpec((RBLK, D_), lambda n: (n, 0)),
            pl.BlockSpec((RBLK, D_), lambda n: (n, 0)),
        ],
        out_shape=[
            jax.ShapeDtypeStruct((N_, D_), jnp.float32),
            jax.ShapeDtypeStruct((N_, D_), jnp.float32),
        ],
    )(x, Wl, Wr, b.reshape(1, D_))


def _combine_mm_stage(parts, pdeg, r_in, Wl, Wr, b):
    def body(p_ref, pd_ref, r_ref, wl_ref, wr_ref, b_ref, z_ref, r2_ref):
        s = p_ref[0] + p_ref[1]
        deg = jnp.maximum(pd_ref[0] + pd_ref[1], 1.0)
        h = _leaky(s / deg + r_ref[...])
        z_ref[...] = jnp.dot(h, wl_ref[...], preferred_element_type=jnp.float32)
        r2_ref[...] = (
            jnp.dot(h, wr_ref[...], preferred_element_type=jnp.float32)
            + b_ref[...]
        )

    return pl.pallas_call(
        body,
        grid=(N_ // RBLK,),
        in_specs=[
            pl.BlockSpec((_NC, RBLK, D_), lambda n: (0, n, 0)),
            pl.BlockSpec((_NC, RBLK, 1), lambda n: (0, n, 0)),
            pl.BlockSpec((RBLK, D_), lambda n: (n, 0)),
            pl.BlockSpec((D_, D_), lambda n: (0, 0)),
            pl.BlockSpec((D_, D_), lambda n: (0, 0)),
            pl.BlockSpec((1, D_), lambda n: (0, 0)),
        ],
        out_specs=[
            pl.BlockSpec((RBLK, D_), lambda n: (n, 0)),
            pl.BlockSpec((RBLK, D_), lambda n: (n, 0)),
        ],
        out_shape=[
            jax.ShapeDtypeStruct((N_, D_), jnp.float32),
            jax.ShapeDtypeStruct((N_, D_), jnp.float32),
        ],
    )(parts, pdeg, r_in, Wl, Wr, b.reshape(1, D_))


def _finish_stage(parts, pdeg, r_in):
    def body(p_ref, pd_ref, r_ref, o_ref):
        s = p_ref[0] + p_ref[1]
        deg = jnp.maximum(pd_ref[0] + pd_ref[1], 1.0)
        o_ref[...] = _leaky(s / deg + r_ref[...])

    return pl.pallas_call(
        body,
        grid=(N_ // RBLK,),
        in_specs=[
            pl.BlockSpec((_NC, RBLK, D_), lambda n: (0, n, 0)),
            pl.BlockSpec((_NC, RBLK, 1), lambda n: (0, n, 0)),
            pl.BlockSpec((RBLK, D_), lambda n: (n, 0)),
        ],
        out_specs=pl.BlockSpec((RBLK, D_), lambda n: (n, 0)),
        out_shape=jax.ShapeDtypeStruct((N_, D_), jnp.float32),
    )(parts, pdeg, r_in)


def _sc_segsum(z, src3, dst3, with_deg):
    n_chunks = PERW // CHUNK
    n_groups = n_chunks // GRP
    rows_per_tile = NPAD // _NSS
    mesh = plsc.VectorSubcoreMesh(core_axis_name="c", subcore_axis_name="s")

    out_type = [jax.ShapeDtypeStruct((_NC, NPAD, D_), jnp.float32)]
    scratch = [
        pltpu.VMEM((GRP, CHUNK), jnp.int32),
        pltpu.VMEM((GRP, CHUNK), jnp.int32),
        pltpu.VMEM((GRP, CHUNK), jnp.int32),
        pltpu.VMEM((GRP, CHUNK), jnp.int32),
        pltpu.VMEM((CHUNK, D_), jnp.float32),
        pltpu.VMEM((CHUNK, D_), jnp.float32),
        pltpu.VMEM((CHUNK, D_), jnp.float32),
        pltpu.VMEM((CHUNK, D_), jnp.float32),
        pltpu.VMEM_SHARED((NPAD, D_), jnp.float32),
        pltpu.SemaphoreType.DMA,
        pltpu.SemaphoreType.DMA,
        pltpu.SemaphoreType.DMA,
        pltpu.SemaphoreType.DMA,
        pltpu.SemaphoreType.DMA,
    ]
    if with_deg:
        out_type.append(jax.ShapeDtypeStruct((_NC, NPAD), jnp.float32))
        scratch += [
            pltpu.VMEM((CHUNK,), jnp.float32),
            pltpu.VMEM((rows_per_tile,), jnp.float32),
            pltpu.VMEM_SHARED((NPAD,), jnp.float32),
        ]

    @functools.partial(
        pl.kernel, mesh=mesh, out_type=out_type, scratch_types=scratch
    )
    def k(z_hbm, src_hbm, dst_hbm, feat_hbm, *rest):
        if with_deg:
            (deg_hbm, srci0, srci1, dsti0, dsti1, rw0, rw1, rw2, rw3, acc_sh,
             sm0, sm1, sm2, sm3, sem_i, ones_v, degrow_v, dacc_sh) = rest
        else:
            (srci0, srci1, dsti0, dsti1, rw0, rw1, rw2, rw3, acc_sh,
             sm0, sm1, sm2, sm3, sem_i) = rest
        rows_a, sem_a = rw0, sm0
        bufs = ((rw0, sm0), (rw1, sm1), (rw2, sm2), (rw3, sm3))
        cid = lax.axis_index("c")
        sid = lax.axis_index("s")
        wid = cid * _NSS + sid

        zv = jnp.zeros((16,), jnp.float32)

        def zrow(r, carry):
            def zcol(j, carry2):
                rows_a[r, pl.ds(j * 16, 16)] = zv
                return carry2

            return lax.fori_loop(0, D_ // 16, zcol, carry)

        lax.fori_loop(0, CHUNK, zrow, 0)
        for t in range(rows_per_tile // CHUNK):
            pltpu.sync_copy(
                rows_a, acc_sh.at[pl.ds(sid * rows_per_tile + t * CHUNK, CHUNK)]
            )
        if with_deg:
            ov = jnp.ones((16,), jnp.float32)

            def fill1(j, carry):
                ones_v[pl.ds(j * 16, 16)] = ov
                return carry

            lax.fori_loop(0, CHUNK // 16, fill1, 0)

            def filld(j, carry):
                degrow_v[pl.ds(j * 16, 16)] = zv
                return carry

            lax.fori_loop(0, rows_per_tile // 16, filld, 0)
            pltpu.sync_copy(
                degrow_v, dacc_sh.at[pl.ds(sid * rows_per_tile, rows_per_tile)]
            )
        plsc.subcore_barrier()

        def fire(si, c, b):
            buf, sem = bufs[b]
            pltpu.async_copy(z_hbm.at[si.at[c]], buf, sem)

        def drain(b):
            buf, sem = bufs[b]
            pltpu.make_async_copy(z_hbm.at[pl.ds(0, CHUNK)], buf, sem).wait()

        def scat(di, c, b):
            buf, _ = bufs[b]
            pltpu.sync_copy(buf, acc_sh.at[di.at[c]], add=True)
            if with_deg:
                pltpu.sync_copy(ones_v, dacc_sh.at[di.at[c]], add=True)

        gbufs = [(srci0, dsti0), (srci1, dsti1)]

        pltpu.sync_copy(src_hbm.at[wid, 0], srci0)
        pltpu.sync_copy(dst_hbm.at[wid, 0], dsti0)
        fire(srci0, 0, 0)
        fire(srci0, 1, 1)

        for g in range(n_groups):
            si, di = gbufs[g % 2]
            sn, dn = gbufs[(g + 1) % 2]
            if g + 1 < n_groups:
                pltpu.async_copy(src_hbm.at[wid, g + 1], sn, sem_i)
                pltpu.async_copy(dst_hbm.at[wid, g + 1], dn, sem_i)

            def body(j, carry, si=si, di=di):
                i = j * 4
                drain(0)
                fire(si, i + 2, 2)
                drain(1)
                fire(si, i + 3, 3)
                scat(di, i, 0)
                scat(di, i + 1, 1)
                drain(2)
                fire(si, i + 4, 0)
                drain(3)
                fire(si, i + 5, 1)
                scat(di, i + 2, 2)
                scat(di, i + 3, 3)
                return carry

            lax.fori_loop(0, GRP // 4 - 1, body, 0)
            i = GRP - 4
            drain(0)
            fire(si, i + 2, 2)
            drain(1)
            fire(si, i + 3, 3)
            scat(di, i, 0)
            scat(di, i + 1, 1)
            if g + 1 < n_groups:
                pltpu.make_async_copy(src_hbm.at[wid, g + 1], sn, sem_i).wait()
                pltpu.make_async_copy(dst_hbm.at[wid, g + 1], dn, sem_i).wait()
                drain(2)
                fire(sn, 0, 0)
                drain(3)
                fire(sn, 1, 1)
            else:
                drain(2)
                drain(3)
            scat(di, i + 2, 2)
            scat(di, i + 3, 3)
        plsc.subcore_barrier()

        for t in range(rows_per_tile // CHUNK):
            off = sid * rows_per_tile + t * CHUNK
            pltpu.sync_copy(acc_sh.at[pl.ds(off, CHUNK)], rows_a)
            pltpu.sync_copy(rows_a, feat_hbm.at[cid, pl.ds(off, CHUNK)])
        if with_deg:
            off = sid * rows_per_tile
            pltpu.sync_copy(dacc_sh.at[pl.ds(off, rows_per_tile)], degrow_v)
            pltpu.sync_copy(degrow_v, deg_hbm.at[cid, pl.ds(off, rows_per_tile)])

    return k(z, src3, dst3)


def kernel(x, edge_index, W1_l, b1, W1_r, W2_l, b2, W2_r):
    nw = _NC * _NSS
    per_w = E_ // nw
    pad = PERW - per_w
    n_chunks = PERW // CHUNK
    pad_dst = (N_ + (jnp.arange(pad, dtype=jnp.int32)[None, :]
                     + 8 * jnp.arange(nw, dtype=jnp.int32)[:, None]) % (NPAD - N_))
    pad_src = ((jnp.arange(pad, dtype=jnp.int32)[None, :] * 37
                + 313 * jnp.arange(nw, dtype=jnp.int32)[:, None]) % N_)
    src3 = jnp.concatenate(
        [edge_index[0].reshape(nw, per_w),
         pad_src.astype(jnp.int32)], axis=1
    ).reshape(nw, n_chunks // GRP, GRP, CHUNK)
    dst3 = jnp.concatenate(
        [edge_index[1].reshape(nw, per_w),
         pad_dst.astype(jnp.int32)], axis=1
    ).reshape(nw, n_chunks // GRP, GRP, CHUNK)
    z1, r1 = _mm_stage(x, W1_l, W1_r, b1)
    p1, pdeg = _sc_segsum(z1, src3, dst3, True)
    pdeg = pdeg.reshape(_NC, NPAD, 1)
    z2, r2 = _combine_mm_stage(p1, pdeg, r1, W2_l, W2_r, b2)
    p2, _ = _sc_segsum(z2, src3, dst3, True)
    return _finish_stage(p2, pdeg, r2)


# speedup vs baseline: 1.0826x; 1.0826x over previous
import functools

import jax
import jax.numpy as jnp
from jax import lax
from jax.experimental import pallas as pl
from jax.experimental.pallas import tpu as pltpu
from jax.experimental.pallas import tpu_sc as plsc

N_ = 10000
E_ = 320000
D_ = 128
NS_ = 0.5

NPAD = 10240
CHUNK = 128
GRP = 16
PERW = 10240
RBLK = 1000

_NC = 2
_NSS = 16


def _leaky(h):
    return jnp.where(h >= 0.0, h, NS_ * h)


def _mm_stage(x, Wl, Wr, b):
    def body(x_ref, wl_ref, wr_ref, b_ref, z_ref, r_ref):
        xv = x_ref[...]
        z_ref[...] = jnp.dot(xv, wl_ref[...], preferred_element_type=jnp.float32)
        r_ref[...] = (
            jnp.dot(xv, wr_ref[...], preferred_element_type=jnp.float32)
            + b_ref[...]
        )

    return pl.pallas_call(
        body,
        grid=(N_ // RBLK,),
        in_specs=[
            pl.BlockSpec((RBLK, D_), lambda n: (n, 0)),
            pl.BlockSpec((D_, D_), lambda n: (0, 0)),
            pl.BlockSpec((D_, D_), lambda n: (0, 0)),
            pl.BlockSpec((1, D_), lambda n: (0, 0)),
        ],
        out_specs=[
            pl.BlockSpec((RBLK, D_), lambda n: (n, 0)),
            pl.BlockSpec((RBLK, D_), lambda n: (n, 0)),
        ],
        out_shape=[
            jax.ShapeDtypeStruct((N_, D_), jnp.float32),
            jax.ShapeDtypeStruct((N_, D_), jnp.float32),
        ],
    )(x, Wl, Wr, b.reshape(1, D_))


def _combine_mm_stage(parts, pdeg, r_in, Wl, Wr, b):
    def body(p_ref, pd_ref, r_ref, wl_ref, wr_ref, b_ref, z_ref, r2_ref):
        s = p_ref[0] + p_ref[1]
        deg = jnp.maximum(pd_ref[0] + pd_ref[1], 1.0)
        h = _leaky(s / deg + r_ref[...])
        z_ref[...] = jnp.dot(h, wl_ref[...], preferred_element_type=jnp.float32)
        r2_ref[...] = (
            jnp.dot(h, wr_ref[...], preferred_element_type=jnp.float32)
            + b_ref[...]
        )

    return pl.pallas_call(
        body,
        grid=(N_ // RBLK,),
        in_specs=[
            pl.BlockSpec((_NC, RBLK, D_), lambda n: (0, n, 0)),
            pl.BlockSpec((_NC, RBLK, 1), lambda n: (0, n, 0)),
            pl.BlockSpec((RBLK, D_), lambda n: (n, 0)),
            pl.BlockSpec((D_, D_), lambda n: (0, 0)),
            pl.BlockSpec((D_, D_), lambda n: (0, 0)),
            pl.BlockSpec((1, D_), lambda n: (0, 0)),
        ],
        out_specs=[
            pl.BlockSpec((RBLK, D_), lambda n: (n, 0)),
            pl.BlockSpec((RBLK, D_), lambda n: (n, 0)),
        ],
        out_shape=[
            jax.ShapeDtypeStruct((N_, D_), jnp.float32),
            jax.ShapeDtypeStruct((N_, D_), jnp.float32),
        ],
    )(parts, pdeg, r_in, Wl, Wr, b.reshape(1, D_))


def _finish_stage(parts, pdeg, r_in):
    def body(p_ref, pd_ref, r_ref, o_ref):
        s = p_ref[0] + p_ref[1]
        deg = jnp.maximum(pd_ref[0] + pd_ref[1], 1.0)
        o_ref[...] = _leaky(s / deg + r_ref[...])

    return pl.pallas_call(
        body,
        grid=(N_ // RBLK,),
        in_specs=[
            pl.BlockSpec((_NC, RBLK, D_), lambda n: (0, n, 0)),
            pl.BlockSpec((_NC, RBLK, 1), lambda n: (0, n, 0)),
            pl.BlockSpec((RBLK, D_), lambda n: (n, 0)),
        ],
        out_specs=pl.BlockSpec((RBLK, D_), lambda n: (n, 0)),
        out_shape=jax.ShapeDtypeStruct((N_, D_), jnp.float32),
    )(parts, pdeg, r_in)


def _sc_segsum(z, src3, dst3, with_deg):
    n_chunks = PERW // CHUNK
    n_groups = n_chunks // GRP
    rows_per_tile = NPAD // _NSS
    mesh = plsc.VectorSubcoreMesh(core_axis_name="c", subcore_axis_name="s")

    out_type = [jax.ShapeDtypeStruct((_NC, NPAD, D_), jnp.float32)]
    scratch = [
        pltpu.VMEM((GRP, CHUNK), jnp.int32),
        pltpu.VMEM((GRP, CHUNK), jnp.int32),
        pltpu.VMEM((GRP, CHUNK), jnp.int32),
        pltpu.VMEM((GRP, CHUNK), jnp.int32),
        pltpu.VMEM((CHUNK, D_), jnp.float32),
        pltpu.VMEM((CHUNK, D_), jnp.float32),
        pltpu.VMEM_SHARED((NPAD, D_), jnp.float32),
        pltpu.SemaphoreType.DMA,
        pltpu.SemaphoreType.DMA,
        pltpu.SemaphoreType.DMA,
    ]
    if with_deg:
        out_type.append(jax.ShapeDtypeStruct((_NC, NPAD), jnp.float32))
        scratch += [
            pltpu.VMEM((CHUNK,), jnp.float32),
            pltpu.VMEM((rows_per_tile,), jnp.float32),
            pltpu.VMEM_SHARED((NPAD,), jnp.float32),
        ]

    @functools.partial(
        pl.kernel, mesh=mesh, out_type=out_type, scratch_types=scratch
    )
    def k(z_hbm, src_hbm, dst_hbm, feat_hbm, *rest):
        if with_deg:
            (deg_hbm, srci0, srci1, dsti0, dsti1, rows_a, rows_b, acc_sh,
             sem_a, sem_b, sem_i, ones_v, degrow_v, dacc_sh) = rest
        else:
            (srci0, srci1, dsti0, dsti1, rows_a, rows_b, acc_sh,
             sem_a, sem_b, sem_i) = rest
        cid = lax.axis_index("c")
        sid = lax.axis_index("s")
        wid = cid * _NSS + sid

        zv = jnp.zeros((16,), jnp.float32)

        def zrow(r, carry):
            def zcol(j, carry2):
                rows_a[r, pl.ds(j * 16, 16)] = zv
                return carry2

            return lax.fori_loop(0, D_ // 16, zcol, carry)

        lax.fori_loop(0, CHUNK, zrow, 0)
        for t in range(rows_per_tile // CHUNK):
            pltpu.sync_copy(
                rows_a, acc_sh.at[pl.ds(sid * rows_per_tile + t * CHUNK, CHUNK)]
            )
        if with_deg:
            ov = jnp.ones((16,), jnp.float32)

            def fill1(j, carry):
                ones_v[pl.ds(j * 16, 16)] = ov
                return carry

            lax.fori_loop(0, CHUNK // 16, fill1, 0)

            def filld(j, carry):
                degrow_v[pl.ds(j * 16, 16)] = zv
                return carry

            lax.fori_loop(0, rows_per_tile // 16, filld, 0)
            pltpu.sync_copy(
                degrow_v, dacc_sh.at[pl.ds(sid * rows_per_tile, rows_per_tile)]
            )
        plsc.subcore_barrier()

        def fire(si, c, buf, sem):
            pltpu.async_copy(z_hbm.at[si.at[c]], buf, sem)

        def drain(buf, sem):
            pltpu.make_async_copy(z_hbm.at[pl.ds(0, CHUNK)], buf, sem).wait()

        def scat(di, c, buf):
            pltpu.sync_copy(buf, acc_sh.at[di.at[c]], add=True)
            if with_deg:
                pltpu.sync_copy(ones_v, dacc_sh.at[di.at[c]], add=True)

        gbufs = [(srci0, dsti0), (srci1, dsti1)]

        pltpu.sync_copy(src_hbm.at[wid, 0], srci0)
        pltpu.sync_copy(dst_hbm.at[wid, 0], dsti0)
        fire(srci0, 0, rows_a, sem_a)

        for g in range(n_groups):
            si, di = gbufs[g % 2]
            sn, dn = gbufs[(g + 1) % 2]
            if g + 1 < n_groups:
                pltpu.async_copy(src_hbm.at[wid, g + 1], sn, sem_i)
                pltpu.async_copy(dst_hbm.at[wid, g + 1], dn, sem_i)

            def body(j, carry, si=si, di=di):
                c = j * 2
                fire(si, c + 1, rows_b, sem_b)
                drain(rows_a, sem_a)
                scat(di, c, rows_a)
                fire(si, c + 2, rows_a, sem_a)
                drain(rows_b, sem_b)
                scat(di, c + 1, rows_b)
                return carry

            lax.fori_loop(0, GRP // 2 - 1, body, 0)
            fire(si, GRP - 1, rows_b, sem_b)
            drain(rows_a, sem_a)
            scat(di, GRP - 2, rows_a)
            if g + 1 < n_groups:
                pltpu.make_async_copy(src_hbm.at[wid, g + 1], sn, sem_i).wait()
                pltpu.make_async_copy(dst_hbm.at[wid, g + 1], dn, sem_i).wait()
                fire(sn, 0, rows_a, sem_a)
            drain(rows_b, sem_b)
            scat(di, GRP - 1, rows_b)
        plsc.subcore_barrier()

        for t in range(rows_per_tile // CHUNK):
            off = sid * rows_per_tile + t * CHUNK
            pltpu.sync_copy(acc_sh.at[pl.ds(off, CHUNK)], rows_a)
            pltpu.sync_copy(rows_a, feat_hbm.at[cid, pl.ds(off, CHUNK)])
        if with_deg:
            off = sid * rows_per_tile
            pltpu.sync_copy(dacc_sh.at[pl.ds(off, rows_per_tile)], degrow_v)
            pltpu.sync_copy(degrow_v, deg_hbm.at[cid, pl.ds(off, rows_per_tile)])

    return k(z, src3, dst3)


def kernel(x, edge_index, W1_l, b1, W1_r, W2_l, b2, W2_r):
    nw = _NC * _NSS
    per_w = E_ // nw
    pad = PERW - per_w
    n_chunks = PERW // CHUNK
    pad_dst = (N_ + (jnp.arange(pad, dtype=jnp.int32)[None, :]
                     + 8 * jnp.arange(nw, dtype=jnp.int32)[:, None]) % (NPAD - N_))
    pad_src = ((jnp.arange(pad, dtype=jnp.int32)[None, :] * 37
                + 313 * jnp.arange(nw, dtype=jnp.int32)[:, None]) % N_)
    src3 = jnp.concatenate(
        [edge_index[0].reshape(nw, per_w),
         pad_src.astype(jnp.int32)], axis=1
    ).reshape(nw, n_chunks // GRP, GRP, CHUNK)
    dst3 = jnp.concatenate(
        [edge_index[1].reshape(nw, per_w),
         pad_dst.astype(jnp.int32)], axis=1
    ).reshape(nw, n_chunks // GRP, GRP, CHUNK)
    z1, r1 = _mm_stage(x, W1_l, W1_r, b1)
    p1, pdeg = _sc_segsum(z1, src3, dst3, True)
    pdeg = pdeg.reshape(_NC, NPAD, 1)
    z2, r2 = _combine_mm_stage(p1, pdeg, r1, W2_l, W2_r, b2)
    p2, _ = _sc_segsum(z2, src3, dst3, True)
    return _finish_stage(p2, pdeg, r2)


# speedup vs baseline: 1.0953x; 1.0118x over previous
import functools

import jax
import jax.numpy as jnp
from jax import lax
from jax.experimental import pallas as pl
from jax.experimental.pallas import tpu as pltpu
from jax.experimental.pallas import tpu_sc as plsc

N_ = 10000
E_ = 320000
D_ = 128
NS_ = 0.5

NPAD = 10240
CHUNK = 128
GRP = 16
PERW = 10240
RBLK = 1000

_NC = 2
_NSS = 16


def _leaky(h):
    return jnp.where(h >= 0.0, h, NS_ * h)


def _mm_stage(x, Wl, Wr, b):
    def body(x_ref, wl_ref, wr_ref, b_ref, z_ref, r_ref):
        xv = x_ref[...]
        z_ref[...] = jnp.dot(xv, wl_ref[...], preferred_element_type=jnp.float32)
        r_ref[...] = (
            jnp.dot(xv, wr_ref[...], preferred_element_type=jnp.float32)
            + b_ref[...]
        )

    return pl.pallas_call(
        body,
        grid=(N_ // RBLK,),
        in_specs=[
            pl.BlockSpec((RBLK, D_), lambda n: (n, 0)),
            pl.BlockSpec((D_, D_), lambda n: (0, 0)),
            pl.BlockSpec((D_, D_), lambda n: (0, 0)),
            pl.BlockSpec((1, D_), lambda n: (0, 0)),
        ],
        out_specs=[
            pl.BlockSpec((RBLK, D_), lambda n: (n, 0)),
            pl.BlockSpec((RBLK, D_), lambda n: (n, 0)),
        ],
        out_shape=[
            jax.ShapeDtypeStruct((N_, D_), jnp.float32),
            jax.ShapeDtypeStruct((N_, D_), jnp.float32),
        ],
    )(x, Wl, Wr, b.reshape(1, D_))


def _combine_mm_stage(parts, pdeg, r_in, Wl, Wr, b):
    def body(p_ref, pd_ref, r_ref, wl_ref, wr_ref, b_ref, z_ref, r2_ref):
        s = p_ref[0] + p_ref[1]
        deg = jnp.maximum(pd_ref[0] + pd_ref[1], 1.0)
        h = _leaky(s / deg + r_ref[...])
        z_ref[...] = jnp.dot(h, wl_ref[...], preferred_element_type=jnp.float32)
        r2_ref[...] = (
            jnp.dot(h, wr_ref[...], preferred_element_type=jnp.float32)
            + b_ref[...]
        )

    return pl.pallas_call(
        body,
        grid=(N_ // RBLK,),
        in_specs=[
            pl.BlockSpec((_NC, RBLK, D_), lambda n: (0, n, 0)),
            pl.BlockSpec((_NC, RBLK, 1), lambda n: (0, n, 0)),
            pl.BlockSpec((RBLK, D_), lambda n: (n, 0)),
            pl.BlockSpec((D_, D_), lambda n: (0, 0)),
            pl.BlockSpec((D_, D_), lambda n: (0, 0)),
            pl.BlockSpec((1, D_), lambda n: (0, 0)),
        ],
        out_specs=[
            pl.BlockSpec((RBLK, D_), lambda n: (n, 0)),
            pl.BlockSpec((RBLK, D_), lambda n: (n, 0)),
        ],
        out_shape=[
            jax.ShapeDtypeStruct((N_, D_), jnp.float32),
            jax.ShapeDtypeStruct((N_, D_), jnp.float32),
        ],
    )(parts, pdeg, r_in, Wl, Wr, b.reshape(1, D_))


def _finish_stage(parts, pdeg, r_in):
    def body(p_ref, pd_ref, r_ref, o_ref):
        s = p_ref[0] + p_ref[1]
        deg = jnp.maximum(pd_ref[0] + pd_ref[1], 1.0)
        o_ref[...] = _leaky(s / deg + r_ref[...])

    return pl.pallas_call(
        body,
        grid=(N_ // RBLK,),
        in_specs=[
            pl.BlockSpec((_NC, RBLK, D_), lambda n: (0, n, 0)),
            pl.BlockSpec((_NC, RBLK, 1), lambda n: (0, n, 0)),
            pl.BlockSpec((RBLK, D_), lambda n: (n, 0)),
        ],
        out_specs=pl.BlockSpec((RBLK, D_), lambda n: (n, 0)),
        out_shape=jax.ShapeDtypeStruct((N_, D_), jnp.float32),
    )(parts, pdeg, r_in)


def _sc_segsum(z, src3, dst3, with_deg):
    n_chunks = PERW // CHUNK
    n_groups = n_chunks // GRP
    rows_per_tile = NPAD // _NSS
    mesh = plsc.VectorSubcoreMesh(core_axis_name="c", subcore_axis_name="s")

    out_type = [jax.ShapeDtypeStruct((_NC, NPAD, D_), jnp.float32)]
    scratch = [
        pltpu.VMEM((GRP, CHUNK), jnp.int32),
        pltpu.VMEM((GRP, CHUNK), jnp.int32),
        pltpu.VMEM((GRP, CHUNK), jnp.int32),
        pltpu.VMEM((GRP, CHUNK), jnp.int32),
        pltpu.VMEM((CHUNK, D_), jnp.float32),
        pltpu.VMEM((CHUNK, D_), jnp.float32),
        pltpu.VMEM_SHARED((NPAD, D_), jnp.float32),
        pltpu.SemaphoreType.DMA,
        pltpu.SemaphoreType.DMA,
        pltpu.SemaphoreType.DMA,
        pltpu.SemaphoreType.DMA,
    ]
    if with_deg:
        out_type.append(jax.ShapeDtypeStruct((_NC, NPAD), jnp.float32))
        scratch += [
            pltpu.VMEM((CHUNK,), jnp.float32),
            pltpu.VMEM((rows_per_tile,), jnp.float32),
            pltpu.VMEM_SHARED((NPAD,), jnp.float32),
        ]

    @functools.partial(
        pl.kernel, mesh=mesh, out_type=out_type, scratch_types=scratch
    )
    def k(z_hbm, src_hbm, dst_hbm, feat_hbm, *rest):
        if with_deg:
            (deg_hbm, srci0, srci1, dsti0, dsti1, rows_a, rows_b, acc_sh,
             sem_a, sem_b, sem_i, sem_d, ones_v, degrow_v, dacc_sh) = rest
        else:
            (srci0, srci1, dsti0, dsti1, rows_a, rows_b, acc_sh,
             sem_a, sem_b, sem_i, sem_d) = rest
        cid = lax.axis_index("c")
        sid = lax.axis_index("s")
        wid = cid * _NSS + sid

        zv = jnp.zeros((16,), jnp.float32)

        def zrow(r, carry):
            def zcol(j, carry2):
                rows_a[r, pl.ds(j * 16, 16)] = zv
                return carry2

            return lax.fori_loop(0, D_ // 16, zcol, carry)

        lax.fori_loop(0, CHUNK, zrow, 0)
        for t in range(rows_per_tile // CHUNK):
            pltpu.sync_copy(
                rows_a, acc_sh.at[pl.ds(sid * rows_per_tile + t * CHUNK, CHUNK)]
            )
        if with_deg:
            ov = jnp.ones((16,), jnp.float32)

            def fill1(j, carry):
                ones_v[pl.ds(j * 16, 16)] = ov
                return carry

            lax.fori_loop(0, CHUNK // 16, fill1, 0)

            def filld(j, carry):
                degrow_v[pl.ds(j * 16, 16)] = zv
                return carry

            lax.fori_loop(0, rows_per_tile // 16, filld, 0)
            pltpu.sync_copy(
                degrow_v, dacc_sh.at[pl.ds(sid * rows_per_tile, rows_per_tile)]
            )
        plsc.subcore_barrier()

        def fire(si, c, buf, sem):
            pltpu.async_copy(z_hbm.at[si.at[c]], buf, sem)

        def drain(buf, sem):
            pltpu.make_async_copy(z_hbm.at[pl.ds(0, CHUNK)], buf, sem).wait()

        def scat(di, c, buf):
            pltpu.sync_copy(buf, acc_sh.at[di.at[c]], add=True)
            if with_deg:
                pltpu.async_copy(ones_v, dacc_sh.at[di.at[c]], sem_d, add=True)

        gbufs = [(srci0, dsti0), (srci1, dsti1)]

        pltpu.sync_copy(src_hbm.at[wid, 0], srci0)
        pltpu.sync_copy(dst_hbm.at[wid, 0], dsti0)
        fire(srci0, 0, rows_a, sem_a)

        for g in range(n_groups):
            si, di = gbufs[g % 2]
            sn, dn = gbufs[(g + 1) % 2]
            if g + 1 < n_groups:
                pltpu.async_copy(src_hbm.at[wid, g + 1], sn, sem_i)
                pltpu.async_copy(dst_hbm.at[wid, g + 1], dn, sem_i)

            def body(j, carry, si=si, di=di):
                c = j * 2
                fire(si, c + 1, rows_b, sem_b)
                drain(rows_a, sem_a)
                scat(di, c, rows_a)
                fire(si, c + 2, rows_a, sem_a)
                drain(rows_b, sem_b)
                scat(di, c + 1, rows_b)
                return carry

            lax.fori_loop(0, GRP // 2 - 1, body, 0)
            fire(si, GRP - 1, rows_b, sem_b)
            drain(rows_a, sem_a)
            scat(di, GRP - 2, rows_a)
            if g + 1 < n_groups:
                pltpu.make_async_copy(src_hbm.at[wid, g + 1], sn, sem_i).wait()
                pltpu.make_async_copy(dst_hbm.at[wid, g + 1], dn, sem_i).wait()
                fire(sn, 0, rows_a, sem_a)
            drain(rows_b, sem_b)
            scat(di, GRP - 1, rows_b)
        if with_deg:
            def ddrain(j, carry):
                pltpu.make_async_copy(
                    deg_hbm.at[cid, pl.ds(0, CHUNK)], ones_v, sem_d).wait()
                return carry

            lax.fori_loop(0, n_chunks, ddrain, 0)
        plsc.subcore_barrier()

        for t in range(rows_per_tile // CHUNK):
            off = sid * rows_per_tile + t * CHUNK
            pltpu.sync_copy(
                acc_sh.at[pl.ds(off, CHUNK)], feat_hbm.at[cid, pl.ds(off, CHUNK)]
            )
        if with_deg:
            off = sid * rows_per_tile
            pltpu.sync_copy(
                dacc_sh.at[pl.ds(off, rows_per_tile)],
                deg_hbm.at[cid, pl.ds(off, rows_per_tile)],
            )

    return k(z, src3, dst3)


def kernel(x, edge_index, W1_l, b1, W1_r, W2_l, b2, W2_r):
    nw = _NC * _NSS
    per_w = E_ // nw
    pad = PERW - per_w
    n_chunks = PERW // CHUNK
    pad_dst = (N_ + (jnp.arange(pad, dtype=jnp.int32)[None, :]
                     + 8 * jnp.arange(nw, dtype=jnp.int32)[:, None]) % (NPAD - N_))
    pad_src = ((jnp.arange(pad, dtype=jnp.int32)[None, :] * 37
                + 313 * jnp.arange(nw, dtype=jnp.int32)[:, None]) % N_)
    src3 = jnp.concatenate(
        [edge_index[0].reshape(nw, per_w),
         pad_src.astype(jnp.int32)], axis=1
    ).reshape(nw, n_chunks // GRP, GRP, CHUNK)
    dst3 = jnp.concatenate(
        [edge_index[1].reshape(nw, per_w),
         pad_dst.astype(jnp.int32)], axis=1
    ).reshape(nw, n_chunks // GRP, GRP, CHUNK)
    z1, r1 = _mm_stage(x, W1_l, W1_r, b1)
    p1, pdeg = _sc_segsum(z1, src3, dst3, True)
    pdeg = pdeg.reshape(_NC, NPAD, 1)
    z2, r2 = _combine_mm_stage(p1, pdeg, r1, W2_l, W2_r, b2)
    p2, _ = _sc_segsum(z2, src3, dst3, True)
    return _finish_stage(p2, pdeg, r2)


# speedup vs baseline: 1.1110x; 1.0143x over previous
import functools

import jax
import jax.numpy as jnp
from jax import lax
from jax.experimental import pallas as pl
from jax.experimental.pallas import tpu as pltpu
from jax.experimental.pallas import tpu_sc as plsc

N_ = 10000
E_ = 320000
D_ = 128
NS_ = 0.5

NPAD = 10240
CHUNK = 128
GRP = 16
PERW = 10240
RBLK = 1000

_NC = 2
_NSS = 16


def _leaky(h):
    return jnp.where(h >= 0.0, h, NS_ * h)


def _mm_stage(x, Wl, Wr, b):
    def body(x_ref, wl_ref, wr_ref, b_ref, z_ref, r_ref):
        xv = x_ref[...]
        z_ref[...] = jnp.dot(xv, wl_ref[...], preferred_element_type=jnp.float32)
        r_ref[...] = (
            jnp.dot(xv, wr_ref[...], preferred_element_type=jnp.float32)
            + b_ref[...]
        )

    return pl.pallas_call(
        body,
        grid=(N_ // RBLK,),
        in_specs=[
            pl.BlockSpec((RBLK, D_), lambda n: (n, 0)),
            pl.BlockSpec((D_, D_), lambda n: (0, 0)),
            pl.BlockSpec((D_, D_), lambda n: (0, 0)),
            pl.BlockSpec((1, D_), lambda n: (0, 0)),
        ],
        out_specs=[
            pl.BlockSpec((RBLK, D_), lambda n: (n, 0)),
            pl.BlockSpec((RBLK, D_), lambda n: (n, 0)),
        ],
        out_shape=[
            jax.ShapeDtypeStruct((N_, D_), jnp.float32),
            jax.ShapeDtypeStruct((N_, D_), jnp.float32),
        ],
    )(x, Wl, Wr, b.reshape(1, D_))


def _combine_mm_stage(parts, pdeg, r_in, Wl, Wr, b):
    def body(p_ref, pd_ref, r_ref, wl_ref, wr_ref, b_ref, z_ref, r2_ref):
        s = p_ref[0] + p_ref[1]
        deg = jnp.maximum(pd_ref[0] + pd_ref[1], 1.0)
        h = _leaky(s / deg + r_ref[...])
        z_ref[...] = jnp.dot(h, wl_ref[...], preferred_element_type=jnp.float32)
        r2_ref[...] = (
            jnp.dot(h, wr_ref[...], preferred_element_type=jnp.float32)
            + b_ref[...]
        )

    return pl.pallas_call(
        body,
        grid=(N_ // RBLK,),
        in_specs=[
            pl.BlockSpec((_NC, RBLK, D_), lambda n: (0, n, 0)),
            pl.BlockSpec((_NC, RBLK, 1), lambda n: (0, n, 0)),
            pl.BlockSpec((RBLK, D_), lambda n: (n, 0)),
            pl.BlockSpec((D_, D_), lambda n: (0, 0)),
            pl.BlockSpec((D_, D_), lambda n: (0, 0)),
            pl.BlockSpec((1, D_), lambda n: (0, 0)),
        ],
        out_specs=[
            pl.BlockSpec((RBLK, D_), lambda n: (n, 0)),
            pl.BlockSpec((RBLK, D_), lambda n: (n, 0)),
        ],
        out_shape=[
            jax.ShapeDtypeStruct((N_, D_), jnp.float32),
            jax.ShapeDtypeStruct((N_, D_), jnp.float32),
        ],
    )(parts, pdeg, r_in, Wl, Wr, b.reshape(1, D_))


def _finish_stage(parts, pdeg, r_in):
    def body(p_ref, pd_ref, r_ref, o_ref):
        s = p_ref[0] + p_ref[1]
        deg = jnp.maximum(pd_ref[0] + pd_ref[1], 1.0)
        o_ref[...] = _leaky(s / deg + r_ref[...])

    return pl.pallas_call(
        body,
        grid=(N_ // RBLK,),
        in_specs=[
            pl.BlockSpec((_NC, RBLK, D_), lambda n: (0, n, 0)),
            pl.BlockSpec((_NC, RBLK, 1), lambda n: (0, n, 0)),
            pl.BlockSpec((RBLK, D_), lambda n: (n, 0)),
        ],
        out_specs=pl.BlockSpec((RBLK, D_), lambda n: (n, 0)),
        out_shape=jax.ShapeDtypeStruct((N_, D_), jnp.float32),
    )(parts, pdeg, r_in)


def _sc_segsum(z, src3, dst3, with_deg):
    n_chunks = PERW // CHUNK
    n_groups = n_chunks // GRP
    rows_per_tile = NPAD // _NSS
    mesh = plsc.VectorSubcoreMesh(core_axis_name="c", subcore_axis_name="s")

    out_type = [jax.ShapeDtypeStruct((_NC, NPAD, D_), jnp.float32)]
    scratch = [
        pltpu.VMEM((GRP, CHUNK), jnp.int32),
        pltpu.VMEM((GRP, CHUNK), jnp.int32),
        pltpu.VMEM((GRP, CHUNK), jnp.int32),
        pltpu.VMEM((GRP, CHUNK), jnp.int32),
        pltpu.VMEM((CHUNK, D_), jnp.float32),
        pltpu.VMEM((CHUNK, D_), jnp.float32),
        pltpu.VMEM_SHARED((NPAD, D_), jnp.float32),
        pltpu.SemaphoreType.DMA,
        pltpu.SemaphoreType.DMA,
        pltpu.SemaphoreType.DMA,
        pltpu.SemaphoreType.DMA,
    ]
    if with_deg:
        out_type.append(jax.ShapeDtypeStruct((_NC, NPAD), jnp.float32))
        scratch += [
            pltpu.VMEM((CHUNK,), jnp.float32),
            pltpu.VMEM((rows_per_tile,), jnp.float32),
            pltpu.VMEM_SHARED((NPAD,), jnp.float32),
        ]

    @functools.partial(
        pl.kernel, mesh=mesh, out_type=out_type, scratch_types=scratch
    )
    def k(z_hbm, src_hbm, dst_hbm, feat_hbm, *rest):
        if with_deg:
            (deg_hbm, srci0, srci1, dsti0, dsti1, rows_a, rows_b, acc_sh,
             sem_a, sem_b, sem_i, sem_d, ones_v, degrow_v, dacc_sh) = rest
        else:
            (srci0, srci1, dsti0, dsti1, rows_a, rows_b, acc_sh,
             sem_a, sem_b, sem_i, sem_d) = rest
        cid = lax.axis_index("c")
        sid = lax.axis_index("s")
        wid = cid * _NSS + sid

        pltpu.async_copy(src_hbm.at[wid, 0], srci0, sem_i)
        pltpu.async_copy(dst_hbm.at[wid, 0], dsti0, sem_i)
        zv = jnp.zeros((16,), jnp.float32)

        def zrow(r, carry):
            def zcol(j, carry2):
                rows_b[r, pl.ds(j * 16, 16)] = zv
                return carry2

            return lax.fori_loop(0, D_ // 16, zcol, carry)

        lax.fori_loop(0, CHUNK, zrow, 0)
        pltpu.make_async_copy(src_hbm.at[wid, 0], srci0, sem_i).wait()
        pltpu.make_async_copy(dst_hbm.at[wid, 0], dsti0, sem_i).wait()
        pltpu.async_copy(z_hbm.at[srci0.at[0]], rows_a, sem_a)
        for t in range(rows_per_tile // CHUNK):
            pltpu.sync_copy(
                rows_b, acc_sh.at[pl.ds(sid * rows_per_tile + t * CHUNK, CHUNK)]
            )
        if with_deg:
            ov = jnp.ones((16,), jnp.float32)

            def fill1(j, carry):
                ones_v[pl.ds(j * 16, 16)] = ov
                return carry

            lax.fori_loop(0, CHUNK // 16, fill1, 0)

            def filld(j, carry):
                degrow_v[pl.ds(j * 16, 16)] = zv
                return carry

            lax.fori_loop(0, rows_per_tile // 16, filld, 0)
            pltpu.sync_copy(
                degrow_v, dacc_sh.at[pl.ds(sid * rows_per_tile, rows_per_tile)]
            )
        plsc.subcore_barrier()

        def fire(si, c, buf, sem):
            pltpu.async_copy(z_hbm.at[si.at[c]], buf, sem)

        def drain(buf, sem):
            pltpu.make_async_copy(z_hbm.at[pl.ds(0, CHUNK)], buf, sem).wait()

        def scat(di, c, buf):
            pltpu.sync_copy(buf, acc_sh.at[di.at[c]], add=True)
            if with_deg:
                pltpu.async_copy(ones_v, dacc_sh.at[di.at[c]], sem_d, add=True)

        gbufs = [(srci0, dsti0), (srci1, dsti1)]

        for g in range(n_groups):
            si, di = gbufs[g % 2]
            sn, dn = gbufs[(g + 1) % 2]
            if g + 1 < n_groups:
                pltpu.async_copy(src_hbm.at[wid, g + 1], sn, sem_i)
                pltpu.async_copy(dst_hbm.at[wid, g + 1], dn, sem_i)

            def body(j, carry, si=si, di=di):
                c = j * 2
                fire(si, c + 1, rows_b, sem_b)
                drain(rows_a, sem_a)
                scat(di, c, rows_a)
                fire(si, c + 2, rows_a, sem_a)
                drain(rows_b, sem_b)
                scat(di, c + 1, rows_b)
                return carry

            lax.fori_loop(0, GRP // 2 - 1, body, 0)
            fire(si, GRP - 1, rows_b, sem_b)
            drain(rows_a, sem_a)
            scat(di, GRP - 2, rows_a)
            if g + 1 < n_groups:
                pltpu.make_async_copy(src_hbm.at[wid, g + 1], sn, sem_i).wait()
                pltpu.make_async_copy(dst_hbm.at[wid, g + 1], dn, sem_i).wait()
                fire(sn, 0, rows_a, sem_a)
            drain(rows_b, sem_b)
            scat(di, GRP - 1, rows_b)
        if with_deg:
            def ddrain(j, carry):
                pltpu.make_async_copy(
                    deg_hbm.at[cid, pl.ds(0, CHUNK)], ones_v, sem_d).wait()
                return carry

            lax.fori_loop(0, n_chunks, ddrain, 0)
        plsc.subcore_barrier()

        for t in range(rows_per_tile // CHUNK):
            off = sid * rows_per_tile + t * CHUNK
            pltpu.sync_copy(
                acc_sh.at[pl.ds(off, CHUNK)], feat_hbm.at[cid, pl.ds(off, CHUNK)]
            )
        if with_deg:
            off = sid * rows_per_tile
            pltpu.sync_copy(
                dacc_sh.at[pl.ds(off, rows_per_tile)],
                deg_hbm.at[cid, pl.ds(off, rows_per_tile)],
            )

    return k(z, src3, dst3)


def kernel(x, edge_index, W1_l, b1, W1_r, W2_l, b2, W2_r):
    nw = _NC * _NSS
    per_w = E_ // nw
    pad = PERW - per_w
    n_chunks = PERW // CHUNK
    pad_dst = (N_ + (jnp.arange(pad, dtype=jnp.int32)[None, :]
                     + 8 * jnp.arange(nw, dtype=jnp.int32)[:, None]) % (NPAD - N_))
    pad_src = ((jnp.arange(pad, dtype=jnp.int32)[None, :] * 37
                + 313 * jnp.arange(nw, dtype=jnp.int32)[:, None]) % N_)
    src3 = jnp.concatenate(
        [edge_index[0].reshape(nw, per_w),
         pad_src.astype(jnp.int32)], axis=1
    ).reshape(nw, n_chunks // GRP, GRP, CHUNK)
    dst3 = jnp.concatenate(
        [edge_index[1].reshape(nw, per_w),
         pad_dst.astype(jnp.int32)], axis=1
    ).reshape(nw, n_chunks // GRP, GRP, CHUNK)
    z1, r1 = _mm_stage(x, W1_l, W1_r, b1)
    p1, pdeg = _sc_segsum(z1, src3, dst3, True)
    pdeg = pdeg.reshape(_NC, NPAD, 1)
    z2, r2 = _combine_mm_stage(p1, pdeg, r1, W2_l, W2_r, b2)
    p2, _ = _sc_segsum(z2, src3, dst3, True)
    return _finish_stage(p2, pdeg, r2)
